# Initial kernel scaffold; baseline (speedup 1.0000x reference)
import jax
import jax.numpy as jnp
from jax.experimental import pallas as pl


def kernel(x, Wr, w1, w2, gamma, beta):
    raise NotImplementedError("write your pallas kernel here")


# speedup vs baseline: 2.8843x; 2.8843x over previous
import functools

import jax
import jax.numpy as jnp
from jax.experimental import pallas as pl
from jax.experimental.pallas import tpu as pltpu

E = 8
D = 1024
H = 512
BT = 512


def _moe_body(x_ref, wr_ref, w1_ref, w2_ref, gamma_ref, beta_ref, o_ref):
    x = x_ref[...]
    logits = jax.lax.dot_general(
        x, wr_ref[...], (((1,), (1,)), ((), ())),
        preferred_element_type=jnp.float32)
    probs = jax.nn.softmax(logits, axis=-1)
    m0 = jnp.max(probs, axis=-1, keepdims=True)
    e0 = jnp.argmax(probs, axis=-1)
    masked = jnp.where(jax.nn.one_hot(e0, E, dtype=jnp.bool_), -jnp.inf, probs)
    m1 = jnp.max(masked, axis=-1, keepdims=True)
    e1 = jnp.argmax(masked, axis=-1)
    denom = m0 + m1
    c0 = m0 / denom
    c1 = m1 / denom

    acc = jnp.zeros((BT, D), dtype=jnp.float32)
    for e in range(E):
        h = jax.lax.dot_general(
            x, w1_ref[e], (((1,), (1,)), ((), ())),
            preferred_element_type=jnp.float32)
        h = h * jax.nn.sigmoid(h)
        y = jax.lax.dot_general(
            h, w2_ref[e], (((1,), (1,)), ((), ())),
            preferred_element_type=jnp.float32)
        coef = jnp.where((e0 == e)[:, None], c0, 0.0) + \
               jnp.where((e1 == e)[:, None], c1, 0.0)
        acc = acc + coef * y

    mean = jnp.mean(acc, axis=-1, keepdims=True)
    var = jnp.mean((acc - mean) ** 2, axis=-1, keepdims=True)
    normed = (acc - mean) * jax.lax.rsqrt(var + 1e-5)
    o_ref[...] = normed * gamma_ref[...][None, :] + beta_ref[...][None, :]


@functools.partial(jax.jit, static_argnames=())
def _moe(x_flat, Wr, w1, w2, gamma, beta):
    T = x_flat.shape[0]
    grid = (T // BT,)
    return pl.pallas_call(
        _moe_body,
        grid=grid,
        in_specs=[
            pl.BlockSpec((BT, D), lambda i: (i, 0)),
            pl.BlockSpec((E, D), lambda i: (0, 0)),
            pl.BlockSpec((E, H, D), lambda i: (0, 0, 0)),
            pl.BlockSpec((E, D, H), lambda i: (0, 0, 0)),
            pl.BlockSpec((D,), lambda i: (0,)),
            pl.BlockSpec((D,), lambda i: (0,)),
        ],
        out_specs=pl.BlockSpec((BT, D), lambda i: (i, 0)),
        out_shape=jax.ShapeDtypeStruct((T, D), jnp.float32),
    )(x_flat, Wr, w1, w2, gamma, beta)


def kernel(x, Wr, w1, w2, gamma, beta):
    B, S, Dm = x.shape
    x_flat = x.reshape(-1, Dm)
    out = _moe(x_flat, Wr, w1, w2, gamma, beta)
    return (out.reshape(B, S, Dm), jnp.asarray(0.0, dtype=jnp.float32))
